# Initial kernel scaffold; baseline (speedup 1.0000x reference)
import jax
import jax.numpy as jnp
from jax.experimental import pallas as pl


def kernel(task_vector, pretensor):
    raise NotImplementedError("write your pallas kernel here")


# speedup vs baseline: 14.8782x; 14.8782x over previous
import functools

import jax
import jax.numpy as jnp
from jax import lax
from jax.experimental import pallas as pl
from jax.experimental.pallas import tpu as pltpu
from jax.experimental.pallas import tpu_sc as plsc

N = 8388608
K_SEL = int(0.01 * N)
NC, NS, L = 2, 16, 16
NW = NC * NS
PER_TILE = N // NW
CHUNK = 8192

B0, B1, B2 = 2048, 4096, 256
SH0, SH1 = 20, 8

_mesh = plsc.VectorSubcoreMesh(
    core_axis_name="c", subcore_axis_name="s", num_cores=NC, num_subcores=NS)
_sc_params = pltpu.CompilerParams(needs_layout_passes=False)


def _wid():
    return lax.axis_index("s") * NC + lax.axis_index("c")


def _abs_bits(x):
    return lax.bitcast_convert_type(x, jnp.int32) & jnp.int32(0x7FFFFFFF)


def _zero_ref(ref, nwords):
    z = jnp.zeros((L,), jnp.int32)

    def body(i, carry):
        ref[pl.ds(i * L, L)] = z
        return carry

    lax.fori_loop(0, nwords // L, body, 0)


def _hist_pass(tv_hbm, buf, hist, nbins, bucket_fn):
    lane = lax.iota(jnp.int32, L)
    ones = jnp.ones((L,), jnp.int32)
    base = _wid() * PER_TILE

    def chunk_body(ci, carry):
        pltpu.sync_copy(tv_hbm.at[pl.ds(base + ci * CHUNK, CHUNK)], buf)

        def inner(j, c2):
            x = buf[pl.ds(j * L, L)]
            bits = _abs_bits(x)
            bkt, mask = bucket_fn(bits)
            idx = lane * nbins + bkt
            plsc.addupdate_scatter(hist, [idx], ones, mask=mask)
            return c2

        lax.fori_loop(0, CHUNK // L, inner, 0)
        return carry

    lax.fori_loop(0, PER_TILE // CHUNK, chunk_body, 0)


def _reduce_ways(hist, hout, nbins):
    def body(i, carry):
        acc = hist[pl.ds(i * L, L)]
        for w in range(1, L):
            acc = acc + hist[pl.ds(w * nbins + i * L, L)]
        hout[pl.ds(i * L, L)] = acc
        return carry

    lax.fori_loop(0, nbins // L, body, 0)


def _accum_global(h_hbm_flat, stage, gbuf, nbins, rows_per_stage):
    _zero_ref(gbuf, nbins)
    n_stages = NW // rows_per_stage

    def stage_body(t, carry):
        pltpu.sync_copy(
            h_hbm_flat.at[pl.ds(t * rows_per_stage * nbins, rows_per_stage * nbins)],
            stage)

        def add_body(i, c2):
            acc = gbuf[pl.ds(i * L, L)]
            for w in range(rows_per_stage):
                acc = acc + stage[pl.ds(w * nbins + i * L, L)]
            gbuf[pl.ds(i * L, L)] = acc
            return c2

        lax.fori_loop(0, nbins // L, add_body, 0)
        return carry

    lax.fori_loop(0, n_stages, stage_body, 0)


def _select(gbuf, nbins, kk):
    lane = lax.iota(jnp.int32, L)
    ngroups = nbins // L
    neg1 = jnp.int32(-1)

    def body(i, carry):
        run, bsel, ksel = carry
        g = ngroups - 1 - i
        v = gbuf[pl.ds(g * L, L)]
        rc = lax.rev(lax.cumsum(lax.rev(v, (0,)), axis=0), (0,))
        above = run + rc
        strictly = above - v
        cond = (above >= kk) & (strictly < kk)
        binidx = g * L + lane
        bsel = jnp.maximum(bsel, jnp.max(jnp.where(cond, binidx, neg1)))
        ksel = jnp.maximum(ksel, jnp.max(jnp.where(cond, kk - strictly, neg1)))
        run = run + jnp.sum(v)
        return (run, bsel, ksel)

    _, bsel, ksel = lax.fori_loop(
        0, ngroups, body, (jnp.int32(0), neg1, neg1))
    return bsel, ksel


def _store_sel(selv, sel_hbm, vals):
    lane = lax.iota(jnp.int32, L)
    out = jnp.zeros((L,), jnp.int32)
    for i, v in enumerate(vals):
        out = jnp.where(lane == i, v, out)
    selv[...] = out

    @pl.when(_wid() == 0)
    def _():
        pltpu.sync_copy(selv, sel_hbm)


def _load_sel(sel_hbm, selv, nvals):
    pltpu.sync_copy(sel_hbm, selv)
    lane = lax.iota(jnp.int32, L)
    v = selv[...]
    big_neg = jnp.int32(-(2 ** 31))
    return [jnp.max(jnp.where(lane == i, v, big_neg)) for i in range(nvals)]


@functools.partial(
    pl.kernel,
    out_type=jax.ShapeDtypeStruct((NW * B0,), jnp.int32),
    mesh=_mesh,
    compiler_params=_sc_params,
    scratch_types=[
        pltpu.VMEM((CHUNK,), jnp.float32),
        pltpu.VMEM((B0 * L,), jnp.int32),
        pltpu.VMEM((B0,), jnp.int32),
    ],
)
def _k_hist0(tv_hbm, h0_hbm, buf, hist, hout):
    _zero_ref(hist, B0 * L)
    _hist_pass(tv_hbm, buf, hist, B0,
               lambda bits: (bits >> SH0, None))
    _reduce_ways(hist, hout, B0)
    pltpu.sync_copy(hout, h0_hbm.at[pl.ds(_wid() * B0, B0)])


@functools.partial(
    pl.kernel,
    out_type=(jax.ShapeDtypeStruct((NW * B1,), jnp.int32),
              jax.ShapeDtypeStruct((L,), jnp.int32)),
    mesh=_mesh,
    compiler_params=_sc_params,
    scratch_types=[
        pltpu.VMEM((CHUNK,), jnp.float32),
        pltpu.VMEM((B1 * L,), jnp.int32),
        pltpu.VMEM((B1,), jnp.int32),
        pltpu.VMEM((8 * B0,), jnp.int32),
        pltpu.VMEM((B0,), jnp.int32),
        pltpu.VMEM((L,), jnp.int32),
    ],
)
def _k_hist1(tv_hbm, h0_hbm, h1_hbm, sel0_hbm, buf, hist, hout, stage, gbuf, selv):
    _accum_global(h0_hbm, stage, gbuf, B0, 8)
    b0, k0 = _select(gbuf, B0, jnp.int32(K_SEL))
    _zero_ref(hist, B1 * L)

    def bucket_fn(bits):
        mask = (bits >> SH0) == b0
        bkt = (bits >> SH1) & jnp.int32(B1 - 1)
        return bkt, mask

    _hist_pass(tv_hbm, buf, hist, B1, bucket_fn)
    _reduce_ways(hist, hout, B1)
    pltpu.sync_copy(hout, h1_hbm.at[pl.ds(_wid() * B1, B1)])
    _store_sel(selv, sel0_hbm, [b0, k0])


@functools.partial(
    pl.kernel,
    out_type=(jax.ShapeDtypeStruct((NW * B2,), jnp.int32),
              jax.ShapeDtypeStruct((L,), jnp.int32)),
    mesh=_mesh,
    compiler_params=_sc_params,
    scratch_types=[
        pltpu.VMEM((CHUNK,), jnp.float32),
        pltpu.VMEM((B2 * L,), jnp.int32),
        pltpu.VMEM((B2,), jnp.int32),
        pltpu.VMEM((8 * B1,), jnp.int32),
        pltpu.VMEM((B1,), jnp.int32),
        pltpu.VMEM((L,), jnp.int32),
    ],
)
def _k_hist2(tv_hbm, h1_hbm, sel0_hbm, h2_hbm, sel1_hbm,
             buf, hist, hout, stage, gbuf, selv):
    b0, k0 = _load_sel(sel0_hbm, selv, 2)
    _accum_global(h1_hbm, stage, gbuf, B1, 8)
    b1, k1 = _select(gbuf, B1, k0)
    _zero_ref(hist, B2 * L)
    prefix = (b0 << 12) | b1

    def bucket_fn(bits):
        mask = (bits >> SH1) == prefix
        bkt = bits & jnp.int32(B2 - 1)
        return bkt, mask

    _hist_pass(tv_hbm, buf, hist, B2, bucket_fn)
    _reduce_ways(hist, hout, B2)
    pltpu.sync_copy(hout, h2_hbm.at[pl.ds(_wid() * B2, B2)])
    _store_sel(selv, sel1_hbm, [b0, b1, k1])


ROWS, COLS = 8192, 1024
BLK_ROWS = 512


def _tc_body(sel_ref, h2_ref, tv_ref, pre_ref, out_ref, thr_ref):
    pid = pl.program_id(0)

    @pl.when(pid == 0)
    def _():
        b0 = sel_ref[0]
        b1 = sel_ref[1]
        k1 = sel_ref[2]
        h2 = h2_ref[...]
        col = lax.broadcasted_iota(jnp.int32, (NW, B2), 1)

        def bs(i, lohi):
            lo, hi = lohi
            mid = (lo + hi + 1) // 2
            cnt = jnp.sum(jnp.where(col >= mid, h2, 0))
            ge = cnt >= k1
            return (jnp.where(ge, mid, lo), jnp.where(ge, hi, mid - 1))

        lo, _hi = lax.fori_loop(0, 8, bs, (jnp.int32(0), jnp.int32(B2 - 1)))
        thr_ref[0] = (b0 << SH0) | (b1 << SH1) | lo

    tb = thr_ref[0]
    tv = tv_ref[...]
    bits = lax.bitcast_convert_type(tv, jnp.int32) & jnp.int32(0x7FFFFFFF)
    bp = jnp.where(bits > tb, jnp.float32(5.0), jnp.float32(-5.0))
    frac = jax.nn.sigmoid(bp)
    out_ref[...] = pre_ref[...] + frac * tv


def _tc_finish(sel1, h2, tv2d, pre2d):
    return pl.pallas_call(
        _tc_body,
        grid=(ROWS // BLK_ROWS,),
        in_specs=[
            pl.BlockSpec(memory_space=pltpu.SMEM),
            pl.BlockSpec((NW, B2), lambda i: (0, 0)),
            pl.BlockSpec((BLK_ROWS, COLS), lambda i: (i, 0)),
            pl.BlockSpec((BLK_ROWS, COLS), lambda i: (i, 0)),
        ],
        out_specs=pl.BlockSpec((BLK_ROWS, COLS), lambda i: (i, 0)),
        out_shape=jax.ShapeDtypeStruct((ROWS, COLS), jnp.float32),
        scratch_shapes=[pltpu.SMEM((1,), jnp.int32)],
    )(sel1, h2, tv2d, pre2d)


@jax.jit
def kernel(task_vector, pretensor):
    tv = task_vector.reshape(-1)
    h0 = _k_hist0(tv)
    h1, sel0 = _k_hist1(tv, h0)
    h2, sel1 = _k_hist2(tv, h1, sel0)
    out2d = _tc_finish(sel1, h2.reshape(NW, B2),
                       tv.reshape(ROWS, COLS), pretensor.reshape(ROWS, COLS))
    return out2d.reshape(task_vector.shape)


# speedup vs baseline: 53.6217x; 3.6040x over previous
import functools

import jax
import jax.numpy as jnp
from jax import lax
from jax.experimental import pallas as pl
from jax.experimental.pallas import tpu as pltpu
from jax.experimental.pallas import tpu_sc as plsc

N = 8388608
K_SEL = int(0.01 * N)
NC, NS, L = 2, 16, 16
NW = NC * NS
PER_TILE = N // NW
CHUNK = 8192

B0 = 32768
B1 = 65536
SH0 = 16

_mesh = plsc.VectorSubcoreMesh(
    core_axis_name="c", subcore_axis_name="s", num_cores=NC, num_subcores=NS)
_sc_params = pltpu.CompilerParams(needs_layout_passes=False)


def _wid():
    return lax.axis_index("s") * NC + lax.axis_index("c")


def _abs_bits(x):
    return lax.bitcast_convert_type(x, jnp.int32) & jnp.int32(0x7FFFFFFF)


def _zero_ref(ref, nwords):
    z = jnp.zeros((L,), jnp.int32)

    @plsc.parallel_loop(0, nwords // L, 1, unroll=8)
    def _(i):
        ref[pl.ds(i * L, L)] = z


def _hist_pass(tv_hbm, buf0, buf1, sem0, sem1, hist, bucket_fn):
    ones = jnp.ones((L,), jnp.int32)
    base = _wid() * PER_TILE
    nchunks = PER_TILE // CHUNK

    def src(ci):
        return tv_hbm.at[pl.ds(base + ci * CHUNK, CHUNK)]

    pltpu.async_copy(src(0), buf0, sem0)
    pltpu.async_copy(src(1), buf1, sem1)

    def process(buf):
        @plsc.parallel_loop(0, CHUNK // L, 1, unroll=8)
        def _(j):
            x = buf[pl.ds(j * L, L)]
            bits = _abs_bits(x)
            bkt, mask = bucket_fn(bits)
            plsc.addupdate_scatter(hist, [bkt], ones, mask=mask)

    def outer(t, carry):
        ci = t * 2
        for b, (buf, sem) in enumerate(((buf0, sem0), (buf1, sem1))):
            pltpu.make_async_copy(src(ci + b), buf, sem).wait()
            process(buf)

            @pl.when(ci + b + 2 < nchunks)
            def _():
                pltpu.async_copy(src(ci + b + 2), buf, sem)

        return carry

    lax.fori_loop(0, nchunks // 2, outer, 0)


def _load_sel(sel_hbm, selv, nvals):
    pltpu.sync_copy(sel_hbm, selv)
    lane = lax.iota(jnp.int32, L)
    v = selv[...]
    big_neg = jnp.int32(-(2 ** 31))
    return [jnp.max(jnp.where(lane == i, v, big_neg)) for i in range(nvals)]


@functools.partial(
    pl.kernel,
    out_type=jax.ShapeDtypeStruct((NW * B0,), jnp.int32),
    mesh=_mesh,
    compiler_params=_sc_params,
    scratch_types=[
        pltpu.VMEM((CHUNK,), jnp.float32),
        pltpu.VMEM((CHUNK,), jnp.float32),
        pltpu.SemaphoreType.DMA,
        pltpu.SemaphoreType.DMA,
        pltpu.VMEM((B0,), jnp.int32),
    ],
)
def _k_hist0(tv_hbm, h0_hbm, buf0, buf1, sem0, sem1, hist):
    _zero_ref(hist, B0)
    _hist_pass(tv_hbm, buf0, buf1, sem0, sem1, hist,
               lambda bits: (bits >> SH0, None))
    pltpu.sync_copy(hist, h0_hbm.at[pl.ds(_wid() * B0, B0)])


@functools.partial(
    pl.kernel,
    out_type=jax.ShapeDtypeStruct((NW * B1,), jnp.int32),
    mesh=_mesh,
    compiler_params=_sc_params,
    scratch_types=[
        pltpu.VMEM((CHUNK,), jnp.float32),
        pltpu.VMEM((CHUNK,), jnp.float32),
        pltpu.SemaphoreType.DMA,
        pltpu.SemaphoreType.DMA,
        pltpu.VMEM((B1,), jnp.int32),
        pltpu.VMEM((L,), jnp.int32),
    ],
)
def _k_hist1(tv_hbm, sel0_hbm, h1_hbm, buf0, buf1, sem0, sem1, hist, selv):
    b0, = _load_sel(sel0_hbm, selv, 1)
    _zero_ref(hist, B1)

    def bucket_fn(bits):
        mask = (bits >> SH0) == b0
        bkt = bits & jnp.int32(B1 - 1)
        return bkt, mask

    _hist_pass(tv_hbm, buf0, buf1, sem0, sem1, hist, bucket_fn)
    pltpu.sync_copy(hist, h1_hbm.at[pl.ds(_wid() * B1, B1)])


def _bsearch(g, flat_idx, kk, nbits):
    def bs(i, lohi):
        lo, hi = lohi
        mid = (lo + hi + 1) // 2
        cnt = jnp.sum(jnp.where(flat_idx >= mid, g, 0))
        ge = cnt >= kk
        return (jnp.where(ge, mid, lo), jnp.where(ge, hi, mid - 1))

    lo, _ = lax.fori_loop(0, nbits, bs, (jnp.int32(0), jnp.int32(2 ** nbits - 1)))
    cnt_gt = jnp.sum(jnp.where(flat_idx > lo, g, 0))
    return lo, cnt_gt


def _sel0_body(h_ref, sel_ref):
    g = jnp.sum(h_ref[...], axis=0)
    rows = B0 // 128
    flat = (lax.broadcasted_iota(jnp.int32, (rows, 128), 0) * 128
            + lax.broadcasted_iota(jnp.int32, (rows, 128), 1))
    b0, cnt_gt = _bsearch(g, flat, jnp.int32(K_SEL), 15)
    sel_ref[0] = b0
    sel_ref[1] = jnp.int32(K_SEL) - cnt_gt
    for i in range(2, L):
        sel_ref[i] = jnp.int32(0)


def _tc_sel0(h0):
    return pl.pallas_call(
        _sel0_body,
        in_specs=[pl.BlockSpec((NW, B0 // 128, 128), lambda: (0, 0, 0))],
        out_specs=pl.BlockSpec(memory_space=pltpu.SMEM),
        out_shape=jax.ShapeDtypeStruct((L,), jnp.int32),
    )(h0)


def _sel1_body(sel0_ref, h_ref, thr_ref):
    g = jnp.sum(h_ref[...], axis=0)
    rows = B1 // 128
    flat = (lax.broadcasted_iota(jnp.int32, (rows, 128), 0) * 128
            + lax.broadcasted_iota(jnp.int32, (rows, 128), 1))
    b1, _ = _bsearch(g, flat, sel0_ref[1], 16)
    thr_ref[0] = (sel0_ref[0] << SH0) | b1
    for i in range(1, L):
        thr_ref[i] = jnp.int32(0)


def _tc_sel1(sel0, h1):
    return pl.pallas_call(
        _sel1_body,
        in_specs=[
            pl.BlockSpec(memory_space=pltpu.SMEM),
            pl.BlockSpec((NW, B1 // 128, 128), lambda: (0, 0, 0)),
        ],
        out_specs=pl.BlockSpec(memory_space=pltpu.SMEM),
        out_shape=jax.ShapeDtypeStruct((L,), jnp.int32),
    )(sel0, h1)


ROWS, COLS = 8192, 1024
BLK_ROWS = 512


def _tc_ew_body(thr_ref, tv_ref, pre_ref, out_ref):
    tb = thr_ref[0]
    tv = tv_ref[...]
    bits = lax.bitcast_convert_type(tv, jnp.int32) & jnp.int32(0x7FFFFFFF)
    bp = jnp.where(bits > tb, jnp.float32(5.0), jnp.float32(-5.0))
    frac = jax.nn.sigmoid(bp)
    out_ref[...] = pre_ref[...] + frac * tv


def _tc_finish(thr, tv2d, pre2d):
    return pl.pallas_call(
        _tc_ew_body,
        grid=(ROWS // BLK_ROWS,),
        in_specs=[
            pl.BlockSpec(memory_space=pltpu.SMEM),
            pl.BlockSpec((BLK_ROWS, COLS), lambda i: (i, 0)),
            pl.BlockSpec((BLK_ROWS, COLS), lambda i: (i, 0)),
        ],
        out_specs=pl.BlockSpec((BLK_ROWS, COLS), lambda i: (i, 0)),
        out_shape=jax.ShapeDtypeStruct((ROWS, COLS), jnp.float32),
    )(thr, tv2d, pre2d)


@jax.jit
def kernel(task_vector, pretensor):
    tv = task_vector.reshape(-1)
    h0 = _k_hist0(tv)
    sel0 = _tc_sel0(h0.reshape(NW, B0 // 128, 128))
    h1 = _k_hist1(tv, sel0)
    thr = _tc_sel1(sel0, h1.reshape(NW, B1 // 128, 128))
    out2d = _tc_finish(thr, tv.reshape(ROWS, COLS), pretensor.reshape(ROWS, COLS))
    return out2d.reshape(task_vector.shape)


# speedup vs baseline: 79.9198x; 1.4904x over previous
import functools

import jax
import jax.numpy as jnp
from jax import lax
from jax.experimental import pallas as pl
from jax.experimental.pallas import tpu as pltpu
from jax.experimental.pallas import tpu_sc as plsc

N = 8388608
K_SEL = int(0.01 * N)
NC, NS, L = 2, 16, 16
NW = NC * NS
PER_TILE = N // NW
CHUNK = 8192

B0 = 32768
B1 = 65536
SH0 = 16

_mesh = plsc.VectorSubcoreMesh(
    core_axis_name="c", subcore_axis_name="s", num_cores=NC, num_subcores=NS)
_sc_params = pltpu.CompilerParams(needs_layout_passes=False)


def _wid():
    return lax.axis_index("s") * NC + lax.axis_index("c")


def _abs_bits(x):
    return lax.bitcast_convert_type(x, jnp.int32) & jnp.int32(0x7FFFFFFF)


def _zero_ref(ref, nwords):
    z = jnp.zeros((L,), jnp.int32)

    @plsc.parallel_loop(0, nwords // L, 1, unroll=8)
    def _(i):
        ref[pl.ds(i * L, L)] = z


def _hist_pass(tv_hbm, buf0, buf1, sem0, sem1, hist, bucket_fn):
    ones = jnp.ones((L,), jnp.int32)
    base = _wid() * PER_TILE
    nchunks = PER_TILE // CHUNK

    def src(ci):
        return tv_hbm.at[pl.ds(base + ci * CHUNK, CHUNK)]

    pltpu.async_copy(src(0), buf0, sem0)
    pltpu.async_copy(src(1), buf1, sem1)

    def process(buf):
        @plsc.parallel_loop(0, CHUNK // L, 1, unroll=8)
        def _(j):
            x = buf[pl.ds(j * L, L)]
            bits = _abs_bits(x)
            bkt, mask = bucket_fn(bits)
            plsc.addupdate_scatter(hist, [bkt], ones, mask=mask)

    def outer(t, carry):
        ci = t * 2
        for b, (buf, sem) in enumerate(((buf0, sem0), (buf1, sem1))):
            pltpu.make_async_copy(src(ci + b), buf, sem).wait()
            process(buf)

            @pl.when(ci + b + 2 < nchunks)
            def _():
                pltpu.async_copy(src(ci + b + 2), buf, sem)

        return carry

    lax.fori_loop(0, nchunks // 2, outer, 0)


def _load_sel(sel_hbm, selv, nvals):
    pltpu.sync_copy(sel_hbm, selv)
    lane = lax.iota(jnp.int32, L)
    v = selv[...]
    big_neg = jnp.int32(-(2 ** 31))
    return [jnp.max(jnp.where(lane == i, v, big_neg)) for i in range(nvals)]


@functools.partial(
    pl.kernel,
    out_type=jax.ShapeDtypeStruct((NW * B0,), jnp.int32),
    mesh=_mesh,
    compiler_params=_sc_params,
    scratch_types=[
        pltpu.VMEM((CHUNK,), jnp.float32),
        pltpu.VMEM((CHUNK,), jnp.float32),
        pltpu.SemaphoreType.DMA,
        pltpu.SemaphoreType.DMA,
        pltpu.VMEM((B0,), jnp.int32),
    ],
)
def _k_hist0(tv_hbm, h0_hbm, buf0, buf1, sem0, sem1, hist):
    _zero_ref(hist, B0)
    _hist_pass(tv_hbm, buf0, buf1, sem0, sem1, hist,
               lambda bits: (bits >> SH0, None))
    pltpu.sync_copy(hist, h0_hbm.at[pl.ds(_wid() * B0, B0)])


@functools.partial(
    pl.kernel,
    out_type=jax.ShapeDtypeStruct((NW * B1,), jnp.int32),
    mesh=_mesh,
    compiler_params=_sc_params,
    scratch_types=[
        pltpu.VMEM((CHUNK,), jnp.float32),
        pltpu.VMEM((CHUNK,), jnp.float32),
        pltpu.SemaphoreType.DMA,
        pltpu.SemaphoreType.DMA,
        pltpu.VMEM((B1,), jnp.int32),
        pltpu.VMEM((L,), jnp.int32),
    ],
)
def _k_hist1(tv_hbm, sel0_hbm, h1_hbm, buf0, buf1, sem0, sem1, hist, selv):
    b0, = _load_sel(sel0_hbm, selv, 1)
    _zero_ref(hist, B1)

    def bucket_fn(bits):
        mask = (bits >> SH0) == b0
        bkt = bits & jnp.int32(B1 - 1)
        return bkt, mask

    _hist_pass(tv_hbm, buf0, buf1, sem0, sem1, hist, bucket_fn)
    pltpu.sync_copy(hist, h1_hbm.at[pl.ds(_wid() * B1, B1)])


def _bsearch(g, flat_idx, kk, nbits):
    def bs(i, lohi):
        lo, hi = lohi
        mid = (lo + hi + 1) // 2
        cnt = jnp.sum(jnp.where(flat_idx >= mid, g, 0))
        ge = cnt >= kk
        return (jnp.where(ge, mid, lo), jnp.where(ge, hi, mid - 1))

    lo, _ = lax.fori_loop(0, nbits, bs, (jnp.int32(0), jnp.int32(2 ** nbits - 1)))
    cnt_gt = jnp.sum(jnp.where(flat_idx > lo, g, 0))
    return lo, cnt_gt


def _sel0_body(h_ref, sel_ref):
    g = jnp.sum(h_ref[...], axis=0)
    rows = B0 // 128
    flat = (lax.broadcasted_iota(jnp.int32, (rows, 128), 0) * 128
            + lax.broadcasted_iota(jnp.int32, (rows, 128), 1))
    b0, cnt_gt = _bsearch(g, flat, jnp.int32(K_SEL), 15)
    sel_ref[0] = b0
    sel_ref[1] = jnp.int32(K_SEL) - cnt_gt
    for i in range(2, L):
        sel_ref[i] = jnp.int32(0)


def _tc_sel0(h0):
    return pl.pallas_call(
        _sel0_body,
        in_specs=[pl.BlockSpec((NW, B0 // 128, 128), lambda: (0, 0, 0))],
        out_specs=pl.BlockSpec(memory_space=pltpu.SMEM),
        out_shape=jax.ShapeDtypeStruct((L,), jnp.int32),
    )(h0)


def _sel1_body(sel0_ref, h_ref, thr_ref):
    g = jnp.sum(h_ref[...], axis=0)
    rows = B1 // 128
    flat = (lax.broadcasted_iota(jnp.int32, (rows, 128), 0) * 128
            + lax.broadcasted_iota(jnp.int32, (rows, 128), 1))
    b1, _ = _bsearch(g, flat, sel0_ref[1], 16)
    thr_ref[0] = (sel0_ref[0] << SH0) | b1
    for i in range(1, L):
        thr_ref[i] = jnp.int32(0)


def _tc_sel1(sel0, h1):
    return pl.pallas_call(
        _sel1_body,
        in_specs=[
            pl.BlockSpec(memory_space=pltpu.SMEM),
            pl.BlockSpec((NW, B1 // 128, 128), lambda: (0, 0, 0)),
        ],
        out_specs=pl.BlockSpec(memory_space=pltpu.SMEM),
        out_shape=jax.ShapeDtypeStruct((L,), jnp.int32),
    )(sel0, h1)


ROWS, COLS = 65536, 128
BLK_ROWS = 4096


def _tc_ew_body(thr_ref, tv_ref, pre_ref, out_ref):
    tb = thr_ref[0]
    tv = tv_ref[...]
    bits = lax.bitcast_convert_type(tv, jnp.int32) & jnp.int32(0x7FFFFFFF)
    bp = jnp.where(bits > tb, jnp.float32(5.0), jnp.float32(-5.0))
    frac = jax.nn.sigmoid(bp)
    out_ref[...] = pre_ref[...] + frac * tv


def _tc_finish(thr, tv2d, pre2d):
    return pl.pallas_call(
        _tc_ew_body,
        grid=(ROWS // BLK_ROWS,),
        in_specs=[
            pl.BlockSpec(memory_space=pltpu.SMEM),
            pl.BlockSpec((BLK_ROWS, COLS), lambda i: (i, 0)),
            pl.BlockSpec((BLK_ROWS, COLS), lambda i: (i, 0)),
        ],
        out_specs=pl.BlockSpec((BLK_ROWS, COLS), lambda i: (i, 0)),
        out_shape=jax.ShapeDtypeStruct((ROWS, COLS), jnp.float32),
    )(thr, tv2d, pre2d)


@jax.jit
def kernel(task_vector, pretensor):
    tv = task_vector.reshape(-1)
    h0 = _k_hist0(tv)
    sel0 = _tc_sel0(h0.reshape(NW, B0 // 128, 128))
    h1 = _k_hist1(tv, sel0)
    thr = _tc_sel1(sel0, h1.reshape(NW, B1 // 128, 128))
    out2d = _tc_finish(thr, tv.reshape(ROWS, COLS), pretensor.reshape(ROWS, COLS))
    return out2d.reshape(task_vector.shape)


# speedup vs baseline: 80.4492x; 1.0066x over previous
import functools

import jax
import jax.numpy as jnp
from jax import lax
from jax.experimental import pallas as pl
from jax.experimental.pallas import tpu as pltpu
from jax.experimental.pallas import tpu_sc as plsc

N = 8388608
K_SEL = int(0.01 * N)
NC, NS, L = 2, 16, 16
NW = NC * NS
PER_TILE = N // NW
CHUNK = 16384

B0 = 32768
B1 = 65536
SH0 = 16

_mesh = plsc.VectorSubcoreMesh(
    core_axis_name="c", subcore_axis_name="s", num_cores=NC, num_subcores=NS)
_sc_params = pltpu.CompilerParams(needs_layout_passes=False)


def _wid():
    return lax.axis_index("s") * NC + lax.axis_index("c")


def _abs_bits(x):
    return lax.bitcast_convert_type(x, jnp.int32) & jnp.int32(0x7FFFFFFF)


def _zero_ref(ref, nwords):
    z = jnp.zeros((L,), jnp.int32)

    @plsc.parallel_loop(0, nwords // L, 1, unroll=8)
    def _(i):
        ref[pl.ds(i * L, L)] = z


def _hist_pass(tv_hbm, buf0, buf1, sem0, sem1, hist, bucket_fn):
    ones = jnp.ones((L,), jnp.int32)
    base = _wid() * PER_TILE
    nchunks = PER_TILE // CHUNK

    def src(ci):
        return tv_hbm.at[pl.ds(base + ci * CHUNK, CHUNK)]

    pltpu.async_copy(src(0), buf0, sem0)
    pltpu.async_copy(src(1), buf1, sem1)

    def process(buf):
        @plsc.parallel_loop(0, CHUNK // L, 1, unroll=8)
        def _(j):
            x = buf[pl.ds(j * L, L)]
            bits = _abs_bits(x)
            bkt, mask = bucket_fn(bits)
            plsc.addupdate_scatter(hist, [bkt], ones, mask=mask)

    def outer(t, carry):
        ci = t * 2
        for b, (buf, sem) in enumerate(((buf0, sem0), (buf1, sem1))):
            pltpu.make_async_copy(src(ci + b), buf, sem).wait()
            process(buf)

            @pl.when(ci + b + 2 < nchunks)
            def _():
                pltpu.async_copy(src(ci + b + 2), buf, sem)

        return carry

    lax.fori_loop(0, nchunks // 2, outer, 0)


def _load_sel(sel_hbm, selv, nvals):
    pltpu.sync_copy(sel_hbm, selv)
    lane = lax.iota(jnp.int32, L)
    v = selv[...]
    big_neg = jnp.int32(-(2 ** 31))
    return [jnp.max(jnp.where(lane == i, v, big_neg)) for i in range(nvals)]


@functools.partial(
    pl.kernel,
    out_type=jax.ShapeDtypeStruct((NW * B0,), jnp.int32),
    mesh=_mesh,
    compiler_params=_sc_params,
    scratch_types=[
        pltpu.VMEM((CHUNK,), jnp.float32),
        pltpu.VMEM((CHUNK,), jnp.float32),
        pltpu.SemaphoreType.DMA,
        pltpu.SemaphoreType.DMA,
        pltpu.VMEM((B0,), jnp.int32),
        pltpu.SemaphoreType.DMA,
    ],
)
def _k_hist0(tv_hbm, zz_hbm, h0_hbm, buf0, buf1, sem0, sem1, hist, zsem):
    pltpu.async_copy(zz_hbm.at[pl.ds(0, B0)], hist, zsem)
    pltpu.make_async_copy(zz_hbm.at[pl.ds(0, B0)], hist, zsem).wait()
    _hist_pass(tv_hbm, buf0, buf1, sem0, sem1, hist,
               lambda bits: (bits >> SH0, None))
    pltpu.sync_copy(hist, h0_hbm.at[pl.ds(_wid() * B0, B0)])


@functools.partial(
    pl.kernel,
    out_type=jax.ShapeDtypeStruct((NW * B1,), jnp.int32),
    mesh=_mesh,
    compiler_params=_sc_params,
    scratch_types=[
        pltpu.VMEM((CHUNK,), jnp.float32),
        pltpu.VMEM((CHUNK,), jnp.float32),
        pltpu.SemaphoreType.DMA,
        pltpu.SemaphoreType.DMA,
        pltpu.VMEM((B1,), jnp.int32),
        pltpu.VMEM((L,), jnp.int32),
        pltpu.SemaphoreType.DMA,
    ],
)
def _k_hist1(tv_hbm, sel0_hbm, zz_hbm, h1_hbm, buf0, buf1, sem0, sem1,
             hist, selv, zsem):
    pltpu.async_copy(zz_hbm, hist, zsem)
    b0, = _load_sel(sel0_hbm, selv, 1)
    pltpu.make_async_copy(zz_hbm, hist, zsem).wait()

    def bucket_fn(bits):
        mask = (bits >> SH0) == b0
        bkt = bits & jnp.int32(B1 - 1)
        return bkt, mask

    _hist_pass(tv_hbm, buf0, buf1, sem0, sem1, hist, bucket_fn)
    pltpu.sync_copy(hist, h1_hbm.at[pl.ds(_wid() * B1, B1)])


def _bsearch(g, flat_idx, kk, nbits):
    def bs(i, lohi):
        lo, hi = lohi
        mid = (lo + hi + 1) // 2
        cnt = jnp.sum(jnp.where(flat_idx >= mid, g, 0))
        ge = cnt >= kk
        return (jnp.where(ge, mid, lo), jnp.where(ge, hi, mid - 1))

    lo, _ = lax.fori_loop(0, nbits, bs, (jnp.int32(0), jnp.int32(2 ** nbits - 1)))
    cnt_gt = jnp.sum(jnp.where(flat_idx > lo, g, 0))
    return lo, cnt_gt


def _sel0_body(h_ref, sel_ref):
    g = jnp.sum(h_ref[...], axis=0)
    rows = B0 // 128
    flat = (lax.broadcasted_iota(jnp.int32, (rows, 128), 0) * 128
            + lax.broadcasted_iota(jnp.int32, (rows, 128), 1))
    b0, cnt_gt = _bsearch(g, flat, jnp.int32(K_SEL), 15)
    sel_ref[0] = b0
    sel_ref[1] = jnp.int32(K_SEL) - cnt_gt
    for i in range(2, L):
        sel_ref[i] = jnp.int32(0)


def _tc_sel0(h0):
    return pl.pallas_call(
        _sel0_body,
        in_specs=[pl.BlockSpec((NW, B0 // 128, 128), lambda: (0, 0, 0))],
        out_specs=pl.BlockSpec(memory_space=pltpu.SMEM),
        out_shape=jax.ShapeDtypeStruct((L,), jnp.int32),
    )(h0)


def _sel1_body(sel0_ref, h_ref, thr_ref):
    g = jnp.sum(h_ref[...], axis=0)
    rows = B1 // 128
    flat = (lax.broadcasted_iota(jnp.int32, (rows, 128), 0) * 128
            + lax.broadcasted_iota(jnp.int32, (rows, 128), 1))
    b1, _ = _bsearch(g, flat, sel0_ref[1], 16)
    thr_ref[0] = (sel0_ref[0] << SH0) | b1
    for i in range(1, L):
        thr_ref[i] = jnp.int32(0)


def _tc_sel1(sel0, h1):
    return pl.pallas_call(
        _sel1_body,
        in_specs=[
            pl.BlockSpec(memory_space=pltpu.SMEM),
            pl.BlockSpec((NW, B1 // 128, 128), lambda: (0, 0, 0)),
        ],
        out_specs=pl.BlockSpec(memory_space=pltpu.SMEM),
        out_shape=jax.ShapeDtypeStruct((L,), jnp.int32),
    )(sel0, h1)


ROWS, COLS = 65536, 128
BLK_ROWS = 4096


def _tc_ew_body(thr_ref, tv_ref, pre_ref, out_ref):
    tb = thr_ref[0]
    tv = tv_ref[...]
    bits = lax.bitcast_convert_type(tv, jnp.int32) & jnp.int32(0x7FFFFFFF)
    bp = jnp.where(bits > tb, jnp.float32(5.0), jnp.float32(-5.0))
    frac = jax.nn.sigmoid(bp)
    out_ref[...] = pre_ref[...] + frac * tv


def _tc_finish(thr, tv2d, pre2d):
    return pl.pallas_call(
        _tc_ew_body,
        grid=(ROWS // BLK_ROWS,),
        in_specs=[
            pl.BlockSpec(memory_space=pltpu.SMEM),
            pl.BlockSpec((BLK_ROWS, COLS), lambda i: (i, 0)),
            pl.BlockSpec((BLK_ROWS, COLS), lambda i: (i, 0)),
        ],
        out_specs=pl.BlockSpec((BLK_ROWS, COLS), lambda i: (i, 0)),
        out_shape=jax.ShapeDtypeStruct((ROWS, COLS), jnp.float32),
    )(thr, tv2d, pre2d)


@jax.jit
def kernel(task_vector, pretensor):
    tv = task_vector.reshape(-1)
    zz = jnp.zeros((B1,), jnp.int32)
    h0 = _k_hist0(tv, zz)
    sel0 = _tc_sel0(h0.reshape(NW, B0 // 128, 128))
    h1 = _k_hist1(tv, sel0, zz)
    thr = _tc_sel1(sel0, h1.reshape(NW, B1 // 128, 128))
    out2d = _tc_finish(thr, tv.reshape(ROWS, COLS), pretensor.reshape(ROWS, COLS))
    return out2d.reshape(task_vector.shape)


# speedup vs baseline: 80.8886x; 1.0055x over previous
import functools

import jax
import jax.numpy as jnp
from jax import lax
from jax.experimental import pallas as pl
from jax.experimental.pallas import tpu as pltpu
from jax.experimental.pallas import tpu_sc as plsc

N = 8388608
K_SEL = int(0.01 * N)
NC, NS, L = 2, 16, 16
NW = NC * NS
PER_TILE = N // NW
CHUNK = 16384

B0 = 32768
B1 = 65536
SH0 = 16

_mesh = plsc.VectorSubcoreMesh(
    core_axis_name="c", subcore_axis_name="s", num_cores=NC, num_subcores=NS)
_sc_params = pltpu.CompilerParams(needs_layout_passes=False)


def _wid():
    return lax.axis_index("s") * NC + lax.axis_index("c")


def _abs_bits(x):
    return lax.bitcast_convert_type(x, jnp.int32) & jnp.int32(0x7FFFFFFF)


def _zero_ref(ref, nwords):
    z = jnp.zeros((L,), jnp.int32)

    @plsc.parallel_loop(0, nwords // L, 1, unroll=8)
    def _(i):
        ref[pl.ds(i * L, L)] = z


def _hist_pass(tv_hbm, buf0, buf1, sem0, sem1, hist, bucket_fn):
    ones = jnp.ones((L,), jnp.int32)
    base = _wid() * PER_TILE
    nchunks = PER_TILE // CHUNK

    def src(ci):
        return tv_hbm.at[pl.ds(base + ci * CHUNK, CHUNK)]

    pltpu.async_copy(src(0), buf0, sem0)
    pltpu.async_copy(src(1), buf1, sem1)

    def process(buf):
        @plsc.parallel_loop(0, CHUNK // L, 1, unroll=8)
        def _(j):
            x = buf[pl.ds(j * L, L)]
            bits = _abs_bits(x)
            bkt, mask = bucket_fn(bits)
            plsc.addupdate_scatter(hist, [bkt], ones, mask=mask)

    def outer(t, carry):
        ci = t * 2
        for b, (buf, sem) in enumerate(((buf0, sem0), (buf1, sem1))):
            pltpu.make_async_copy(src(ci + b), buf, sem).wait()
            process(buf)

            @pl.when(ci + b + 2 < nchunks)
            def _():
                pltpu.async_copy(src(ci + b + 2), buf, sem)

        return carry

    lax.fori_loop(0, nchunks // 2, outer, 0)


def _load_sel(sel_hbm, selv, nvals):
    pltpu.sync_copy(sel_hbm, selv)
    lane = lax.iota(jnp.int32, L)
    v = selv[...]
    big_neg = jnp.int32(-(2 ** 31))
    return [jnp.max(jnp.where(lane == i, v, big_neg)) for i in range(nvals)]


@functools.partial(
    pl.kernel,
    out_type=jax.ShapeDtypeStruct((NW * B0,), jnp.int32),
    mesh=_mesh,
    compiler_params=_sc_params,
    scratch_types=[
        pltpu.VMEM((CHUNK,), jnp.float32),
        pltpu.VMEM((CHUNK,), jnp.float32),
        pltpu.SemaphoreType.DMA,
        pltpu.SemaphoreType.DMA,
        pltpu.VMEM((B0,), jnp.int32),
        pltpu.SemaphoreType.DMA,
    ],
)
def _k_hist0(tv_hbm, zz_hbm, h0_hbm, buf0, buf1, sem0, sem1, hist, zsem):
    pltpu.async_copy(zz_hbm.at[pl.ds(0, B0)], hist, zsem)
    pltpu.make_async_copy(zz_hbm.at[pl.ds(0, B0)], hist, zsem).wait()
    _hist_pass(tv_hbm, buf0, buf1, sem0, sem1, hist,
               lambda bits: (bits >> SH0, None))
    pltpu.sync_copy(hist, h0_hbm.at[pl.ds(_wid() * B0, B0)])


@functools.partial(
    pl.kernel,
    out_type=jax.ShapeDtypeStruct((NW * B1,), jnp.int32),
    mesh=_mesh,
    compiler_params=_sc_params,
    scratch_types=[
        pltpu.VMEM((CHUNK,), jnp.float32),
        pltpu.VMEM((CHUNK,), jnp.float32),
        pltpu.SemaphoreType.DMA,
        pltpu.SemaphoreType.DMA,
        pltpu.VMEM((B1,), jnp.int32),
        pltpu.VMEM((L,), jnp.int32),
        pltpu.SemaphoreType.DMA,
    ],
)
def _k_hist1(tv_hbm, sel0_hbm, zz_hbm, h1_hbm, buf0, buf1, sem0, sem1,
             hist, selv, zsem):
    pltpu.async_copy(zz_hbm, hist, zsem)
    b0, = _load_sel(sel0_hbm, selv, 1)
    pltpu.make_async_copy(zz_hbm, hist, zsem).wait()

    def bucket_fn(bits):
        mask = (bits >> SH0) == b0
        bkt = bits & jnp.int32(B1 - 1)
        return bkt, mask

    _hist_pass(tv_hbm, buf0, buf1, sem0, sem1, hist, bucket_fn)
    pltpu.sync_copy(hist, h1_hbm.at[pl.ds(_wid() * B1, B1)])


def _bsearch(g, flat_idx, kk, nbits):
    def bs(i, lohi):
        lo, hi = lohi
        mid = (lo + hi + 1) // 2
        cnt = jnp.sum(jnp.where(flat_idx >= mid, g, 0))
        ge = cnt >= kk
        return (jnp.where(ge, mid, lo), jnp.where(ge, hi, mid - 1))

    lo, _ = lax.fori_loop(0, nbits, bs, (jnp.int32(0), jnp.int32(2 ** nbits - 1)))
    cnt_gt = jnp.sum(jnp.where(flat_idx > lo, g, 0))
    return lo, cnt_gt


def _sel0_body(h_ref, sel_ref):
    g = jnp.sum(h_ref[...], axis=0)
    rows = B0 // 128
    flat = (lax.broadcasted_iota(jnp.int32, (rows, 128), 0) * 128
            + lax.broadcasted_iota(jnp.int32, (rows, 128), 1))
    b0, cnt_gt = _bsearch(g, flat, jnp.int32(K_SEL), 15)
    sel_ref[0] = b0
    sel_ref[1] = jnp.int32(K_SEL) - cnt_gt
    for i in range(2, L):
        sel_ref[i] = jnp.int32(0)


def _tc_sel0(h0):
    return pl.pallas_call(
        _sel0_body,
        in_specs=[pl.BlockSpec((NW, B0 // 128, 128), lambda: (0, 0, 0))],
        out_specs=pl.BlockSpec(memory_space=pltpu.SMEM),
        out_shape=jax.ShapeDtypeStruct((L,), jnp.int32),
    )(h0)


ROWS, COLS = 65536, 128
BLK_ROWS = 4096


def _tc_ew_body(sel0_ref, h1_hbm, tv_ref, pre_ref, out_ref, hbuf, hsem, thr_ref):
    pid = pl.program_id(0)

    @pl.when(pid == 0)
    def _():
        pltpu.make_async_copy(h1_hbm, hbuf, hsem).start()
        pltpu.make_async_copy(h1_hbm, hbuf, hsem).wait()
        g = jnp.sum(hbuf[...], axis=0)
        rows = B1 // 128
        flat = (lax.broadcasted_iota(jnp.int32, (rows, 128), 0) * 128
                + lax.broadcasted_iota(jnp.int32, (rows, 128), 1))
        b1, _ = _bsearch(g, flat, sel0_ref[1], 16)
        thr_ref[0] = (sel0_ref[0] << SH0) | b1

    tb = thr_ref[0]
    tv = tv_ref[...]
    bits = lax.bitcast_convert_type(tv, jnp.int32) & jnp.int32(0x7FFFFFFF)
    bp = jnp.where(bits > tb, jnp.float32(5.0), jnp.float32(-5.0))
    frac = jax.nn.sigmoid(bp)
    out_ref[...] = pre_ref[...] + frac * tv


def _tc_finish(sel0, h1, tv2d, pre2d):
    return pl.pallas_call(
        _tc_ew_body,
        grid=(ROWS // BLK_ROWS,),
        in_specs=[
            pl.BlockSpec(memory_space=pltpu.SMEM),
            pl.BlockSpec(memory_space=pl.ANY),
            pl.BlockSpec((BLK_ROWS, COLS), lambda i: (i, 0)),
            pl.BlockSpec((BLK_ROWS, COLS), lambda i: (i, 0)),
        ],
        out_specs=pl.BlockSpec((BLK_ROWS, COLS), lambda i: (i, 0)),
        out_shape=jax.ShapeDtypeStruct((ROWS, COLS), jnp.float32),
        scratch_shapes=[
            pltpu.VMEM((NW, B1 // 128, 128), jnp.int32),
            pltpu.SemaphoreType.DMA,
            pltpu.SMEM((1,), jnp.int32),
        ],
    )(sel0, h1, tv2d, pre2d)


@jax.jit
def kernel(task_vector, pretensor):
    tv = task_vector.reshape(-1)
    zz = jnp.zeros((B1,), jnp.int32)
    h0 = _k_hist0(tv, zz)
    sel0 = _tc_sel0(h0.reshape(NW, B0 // 128, 128))
    h1 = _k_hist1(tv, sel0, zz)
    out2d = _tc_finish(sel0, h1.reshape(NW, B1 // 128, 128),
                       tv.reshape(ROWS, COLS), pretensor.reshape(ROWS, COLS))
    return out2d.reshape(task_vector.shape)
